# Initial kernel scaffold; baseline (speedup 1.0000x reference)
import jax
import jax.numpy as jnp
from jax.experimental import pallas as pl


def kernel(x, edge_index, W1, a1_src, a1_dst, b1, W2, a2_src, a2_dst, b2):
    raise NotImplementedError("write your pallas kernel here")


# speedup vs baseline: 1.1657x; 1.1657x over previous
import functools

import jax
import jax.numpy as jnp
from jax.experimental import pallas as pl
from jax.experimental.pallas import tpu as pltpu

N_NODES = 10000
D = 128
ROW_BLOCK = 2000


def _mm_body(x_ref, w_ref, h_ref):
    h_ref[...] = jnp.dot(x_ref[...], w_ref[...],
                         preferred_element_type=jnp.float32)


def _matmul(x, w):
    n = x.shape[0]
    return pl.pallas_call(
        _mm_body,
        grid=(n // ROW_BLOCK,),
        in_specs=[
            pl.BlockSpec((ROW_BLOCK, D), lambda i: (i, 0)),
            pl.BlockSpec((D, D), lambda i: (0, 0)),
        ],
        out_specs=pl.BlockSpec((ROW_BLOCK, D), lambda i: (i, 0)),
        out_shape=jax.ShapeDtypeStruct((n, D), jnp.float32),
    )(x, w)


def _gat_layer(x, src, dst, W, a_s, a_d, b):
    h = _matmul(x, W)
    alpha_src = h @ a_s
    alpha_dst = h @ a_d
    e = jax.nn.leaky_relu(alpha_src[src] + alpha_dst[dst], negative_slope=0.2)
    e_max = jax.ops.segment_max(e, dst, num_segments=N_NODES)
    e = e - e_max[dst]
    exp_e = jnp.exp(e)
    denom = jax.ops.segment_sum(exp_e, dst, num_segments=N_NODES)
    attn = exp_e / (denom[dst] + 1e-16)
    out = jax.ops.segment_sum(h[src] * attn[:, None], dst, num_segments=N_NODES)
    return out + b


def kernel(x, edge_index, W1, a1_src, a1_dst, b1, W2, a2_src, a2_dst, b2):
    src = edge_index[0]
    dst = edge_index[1]
    h1 = jax.nn.elu(_gat_layer(x, src, dst, W1, a1_src, a1_dst, b1))
    out = _gat_layer(h1, src, dst, W2, a2_src, a2_dst, b2)
    return out


# speedup vs baseline: 18.7207x; 16.0593x over previous
import functools

import jax
import jax.numpy as jnp
from jax import lax
from jax.experimental import pallas as pl
from jax.experimental.pallas import tpu as pltpu
from jax.experimental.pallas import tpu_sc as plsc

N_NODES = 10000
N_PAD = 10240
D = 128
N_EDGES = 320000
E_PAD = 327680
EROWS = 2560
ROWS_PER_WORKER = 80
STRIPE = 640


def _dense_body(x_ref, w_ref, as_ref, ad_ref, h_ref, als_ref, ald_ref,
                cm_ref):
    h = jnp.dot(x_ref[...], w_ref[...], preferred_element_type=jnp.float32)
    h_ref[...] = h
    als = jnp.sum(h * as_ref[...][None, :], axis=1)
    ald = jnp.sum(h * ad_ref[...][None, :], axis=1)
    als_ref[...] = als
    ald_ref[...] = ald
    bm = jnp.stack([jnp.max(als), jnp.max(ald)])

    @pl.when(pl.program_id(0) == 0)
    def _():
        cm_ref[...] = bm

    @pl.when(pl.program_id(0) != 0)
    def _():
        cm_ref[...] = jnp.maximum(cm_ref[...], bm)


def _dense(x, W, a_s, a_d):
    return pl.pallas_call(
        _dense_body,
        grid=(5,),
        in_specs=[
            pl.BlockSpec((2048, D), lambda i: (i, 0)),
            pl.BlockSpec((D, D), lambda i: (0, 0)),
            pl.BlockSpec((D,), lambda i: (0,)),
            pl.BlockSpec((D,), lambda i: (0,)),
        ],
        out_specs=[
            pl.BlockSpec((2048, D), lambda i: (i, 0)),
            pl.BlockSpec((2048,), lambda i: (i,)),
            pl.BlockSpec((2048,), lambda i: (i,)),
            pl.BlockSpec((2,), lambda i: (0,)),
        ],
        out_shape=[
            jax.ShapeDtypeStruct((N_PAD, D), jnp.float32),
            jax.ShapeDtypeStruct((N_PAD,), jnp.float32),
            jax.ShapeDtypeStruct((N_PAD,), jnp.float32),
            jax.ShapeDtypeStruct((2,), jnp.float32),
        ],
    )(x, W, a_s, a_d)


def _dense2_body(p0_ref, p1_ref, d0_ref, d1_ref, b_ref, w_ref, as_ref,
                 ad_ref, h_ref, als_ref, ald_ref, cm_ref):
    inv = 1.0 / (d0_ref[...] + d1_ref[...] + 1e-16)
    t = (p0_ref[...] + p1_ref[...]) * inv[:, None] + b_ref[...][None, :]
    t = jnp.where(t > 0, t, jnp.exp(t) - 1.0)
    h = jnp.dot(t, w_ref[...], preferred_element_type=jnp.float32)
    h_ref[...] = h
    als = jnp.sum(h * as_ref[...][None, :], axis=1)
    ald = jnp.sum(h * ad_ref[...][None, :], axis=1)
    als_ref[...] = als
    ald_ref[...] = ald
    bm = jnp.stack([jnp.max(als), jnp.max(ald)])

    @pl.when(pl.program_id(0) == 0)
    def _():
        cm_ref[...] = bm

    @pl.when(pl.program_id(0) != 0)
    def _():
        cm_ref[...] = jnp.maximum(cm_ref[...], bm)


def _dense2(p, d0, d1, b, W, a_s, a_d):
    return pl.pallas_call(
        _dense2_body,
        grid=(5,),
        in_specs=[
            pl.BlockSpec((2048, D), lambda i: (i, 0)),
            pl.BlockSpec((2048, D), lambda i: (i + 5, 0)),
            pl.BlockSpec((2048,), lambda i: (i,)),
            pl.BlockSpec((2048,), lambda i: (i,)),
            pl.BlockSpec((D,), lambda i: (0,)),
            pl.BlockSpec((D, D), lambda i: (0, 0)),
            pl.BlockSpec((D,), lambda i: (0,)),
            pl.BlockSpec((D,), lambda i: (0,)),
        ],
        out_specs=[
            pl.BlockSpec((2048, D), lambda i: (i, 0)),
            pl.BlockSpec((2048,), lambda i: (i,)),
            pl.BlockSpec((2048,), lambda i: (i,)),
            pl.BlockSpec((2,), lambda i: (0,)),
        ],
        out_shape=[
            jax.ShapeDtypeStruct((N_PAD, D), jnp.float32),
            jax.ShapeDtypeStruct((N_PAD,), jnp.float32),
            jax.ShapeDtypeStruct((N_PAD,), jnp.float32),
            jax.ShapeDtypeStruct((2,), jnp.float32),
        ],
    )(p, p, d0, d1, b, W, a_s, a_d)


def _final_body(p0_ref, p1_ref, d0_ref, d1_ref, b_ref, o_ref):
    inv = 1.0 / (d0_ref[...] + d1_ref[...] + 1e-16)
    o_ref[...] = ((p0_ref[...] + p1_ref[...]) * inv[:, None]
                  + b_ref[...][None, :])


def _final(p, d0, d1, b):
    return pl.pallas_call(
        _final_body,
        grid=(5,),
        in_specs=[
            pl.BlockSpec((2048, D), lambda i: (i, 0)),
            pl.BlockSpec((2048, D), lambda i: (i + 5, 0)),
            pl.BlockSpec((2048,), lambda i: (i,)),
            pl.BlockSpec((2048,), lambda i: (i,)),
            pl.BlockSpec((D,), lambda i: (0,)),
        ],
        out_specs=pl.BlockSpec((2048, D), lambda i: (i, 0)),
        out_shape=jax.ShapeDtypeStruct((N_PAD, D), jnp.float32),
    )(p, p, d0, d1, b)


_MESH = plsc.VectorSubcoreMesh(core_axis_name="c", subcore_axis_name="s",
                               num_cores=2)


@functools.partial(
    pl.kernel,
    mesh=_MESH,
    out_type=[
        jax.ShapeDtypeStruct((2 * N_PAD, D), jnp.float32),
        jax.ShapeDtypeStruct((N_PAD,), jnp.float32),
        jax.ShapeDtypeStruct((N_PAD,), jnp.float32),
    ],
    scratch_types=[
        pltpu.VMEM((ROWS_PER_WORKER, 128), jnp.int32),
        pltpu.VMEM((ROWS_PER_WORKER, 128), jnp.int32),
        pltpu.VMEM((N_PAD,), jnp.float32),
        pltpu.VMEM((16,), jnp.float32),
        pltpu.VMEM((128,), jnp.float32),
        pltpu.VMEM((128,), jnp.float32),
        pltpu.VMEM((128,), jnp.float32),
        pltpu.VMEM((128, D), jnp.float32),
        pltpu.VMEM_SHARED((N_PAD, D), jnp.float32),
        pltpu.VMEM_SHARED((N_PAD,), jnp.float32),
        pltpu.SemaphoreType.DMA,
        pltpu.SemaphoreType.DMA,
        pltpu.SemaphoreType.DMA,
    ],
)
def _sc_edge(h_hbm, als_hbm, ald_hbm, src_hbm, dst_hbm, cvec_hbm,
             out_hbm, den0_hbm, den1_hbm,
             srcT, dstT, zden, cv, gA, gB, ebuf, rowbuf,
             shared_out, shared_den, sem, sem2, sem3):
    cid = lax.axis_index("c")
    sid = lax.axis_index("s")
    wid = cid * 16 + sid
    z16 = jnp.zeros((16,), jnp.float32)

    def _zrow(r, carry):
        for g in range(8):
            rowbuf[r, pl.ds(g * 16, 16)] = z16
        return carry
    lax.fori_loop(0, 128, _zrow, 0)

    def _zden(i, carry):
        zden[pl.ds(i * 16, 16)] = z16
        return carry
    lax.fori_loop(0, N_PAD // 16, _zden, 0)

    r0 = sid * STRIPE
    for i in range(STRIPE // 128):
        pltpu.sync_copy(rowbuf, shared_out.at[pl.ds(r0 + i * 128, 128)])

    @pl.when(sid == 0)
    def _():
        pltpu.sync_copy(zden, shared_den)

    pltpu.sync_copy(cvec_hbm, cv)
    arow = wid * ROWS_PER_WORKER
    pltpu.sync_copy(src_hbm.at[pl.ds(arow, ROWS_PER_WORKER)], srcT)
    pltpu.sync_copy(dst_hbm.at[pl.ds(arow, ROWS_PER_WORKER)], dstT)
    c = cv[...]

    plsc.subcore_barrier()

    def _row(t, carry):
        cph = pltpu.async_copy(h_hbm.at[srcT.at[t]], rowbuf, sem)
        cpa = pltpu.async_copy(als_hbm.at[srcT.at[t]], gA, sem2)
        cpb = pltpu.async_copy(ald_hbm.at[dstT.at[t]], gB, sem3)
        cpa.wait()
        cpb.wait()
        for v in range(8):
            sl = pl.ds(v * 16, 16)
            sm = gA[sl] + gB[sl]
            e = jnp.where(sm >= 0, sm, 0.2 * sm) - c
            ebuf[sl] = jnp.exp(e)
        pltpu.sync_copy(ebuf, shared_den.at[dstT.at[t]], add=True)
        cph.wait()
        for k in range(8):
            ev = ebuf[pl.ds(k * 16, 16)]
            for l in range(16):
                j = k * 16 + l
                w = ev[l]
                for g in range(8):
                    sl = pl.ds(g * 16, 16)
                    rowbuf[j, sl] = rowbuf[j, sl] * w
        pltpu.sync_copy(rowbuf, shared_out.at[dstT.at[t]], add=True)
        return carry
    lax.fori_loop(0, ROWS_PER_WORKER, _row, 0)

    plsc.subcore_barrier()

    pltpu.sync_copy(shared_out.at[pl.ds(r0, STRIPE)],
                    out_hbm.at[pl.ds(cid * N_PAD + r0, STRIPE)])

    @pl.when(jnp.logical_and(sid == 0, cid == 0))
    def _():
        pltpu.sync_copy(shared_den, den0_hbm)

    @pl.when(jnp.logical_and(sid == 0, cid == 1))
    def _():
        pltpu.sync_copy(shared_den, den1_hbm)


def kernel(x, edge_index, W1, a1_src, a1_dst, b1, W2, a2_src, a2_dst, b2):
    x_pad = jnp.pad(x, ((0, N_PAD - N_NODES), (0, 0)))
    src = jnp.pad(edge_index[0], (0, E_PAD - N_EDGES))
    dst = jnp.pad(edge_index[1], (0, E_PAD - N_EDGES),
                  constant_values=N_PAD - 1)
    src2d = src.reshape(EROWS, 128)
    dst2d = dst.reshape(EROWS, 128)

    h1, als1, ald1, cm1 = _dense(x_pad, W1, a1_src, a1_dst)
    c1 = cm1[0] + cm1[1]
    c1 = jnp.where(c1 >= 0, c1, 0.2 * c1)
    p1, d10, d11 = _sc_edge(h1, als1, ald1, src2d, dst2d,
                            jnp.full((16,), c1, jnp.float32))
    h2, als2, ald2, cm2 = _dense2(p1, d10, d11, b1, W2, a2_src, a2_dst)
    c2 = cm2[0] + cm2[1]
    c2 = jnp.where(c2 >= 0, c2, 0.2 * c2)
    p2, d20, d21 = _sc_edge(h2, als2, ald2, src2d, dst2d,
                            jnp.full((16,), c2, jnp.float32))
    return _final(p2, d20, d21, b2)[:N_NODES]
